# Initial kernel scaffold; baseline (speedup 1.0000x reference)
import functools

import jax
import jax.numpy as jnp
from jax import lax
from jax.experimental import pallas as pl
from jax.experimental.pallas import tpu as pltpu
from jax.experimental.pallas import tpu_sc as plsc

_NC, _NS = 2, 16
_NW = _NC * _NS
_CHUNK = 128


@functools.cache
def _make_kernel(n_rows: int, d: int):
    rows_per_w = n_rows // _NW
    n_chunks = rows_per_w // _CHUNK
    mesh = plsc.VectorSubcoreMesh(
        core_axis_name="c", subcore_axis_name="s",
        num_cores=_NC, num_subcores=_NS,
    )

    @functools.partial(
        pl.kernel,
        out_type=jax.ShapeDtypeStruct((n_rows, d), jnp.float32),
        mesh=mesh,
        scratch_types=[
            pltpu.VMEM((n_chunks, _CHUNK), jnp.int32),
            pltpu.VMEM((_CHUNK, d), jnp.float32),
            pltpu.SemaphoreType.DMA,
        ],
    )
    def k(idx_hbm, table_hbm, out_hbm, idx_v, buf, sem):
        wid = lax.axis_index("s") * _NC + lax.axis_index("c")
        chunk0 = wid * n_chunks
        pltpu.sync_copy(idx_hbm.at[pl.ds(chunk0, n_chunks)], idx_v)

        @pl.loop(0, n_chunks)
        def _(j):
            pltpu.async_copy(table_hbm.at[idx_v.at[j]], buf, sem).wait()
            pltpu.sync_copy(buf, out_hbm.at[pl.ds((chunk0 + j) * _CHUNK, _CHUNK)])

    return k


def kernel(indices, table):
    b, h = indices.shape
    _, d = table.shape
    n = b * h
    idx = indices.reshape(n // _CHUNK, _CHUNK).astype(jnp.int32)
    out = _make_kernel(n, d)(idx, table)
    return out.reshape(b, h, d)


# speedup vs baseline: 4.0874x; 4.0874x over previous
import functools

import jax
import jax.numpy as jnp
from jax import lax
from jax.experimental import pallas as pl
from jax.experimental.pallas import tpu as pltpu
from jax.experimental.pallas import tpu_sc as plsc

_NC, _NS = 2, 16
_NW = _NC * _NS
_CHUNK = 128


@functools.cache
def _make_kernel(n_rows: int, d: int):
    rows_per_w = n_rows // _NW
    n_chunks = rows_per_w // _CHUNK
    mesh = plsc.VectorSubcoreMesh(
        core_axis_name="c", subcore_axis_name="s",
        num_cores=_NC, num_subcores=_NS,
    )

    @functools.partial(
        pl.kernel,
        out_type=jax.ShapeDtypeStruct((n_rows, d), jnp.float32),
        mesh=mesh,
        scratch_types=[
            pltpu.VMEM((n_chunks, _CHUNK), jnp.int32),
            pltpu.VMEM((_CHUNK, d), jnp.float32),
            pltpu.SemaphoreType.DMA,
        ],
        compiler_params=pltpu.CompilerParams(use_tc_tiling_on_sc=False),
    )
    def k(idx_hbm, table_hbm, out_hbm, idx_v, buf, sem):
        wid = lax.axis_index("s") * _NC + lax.axis_index("c")
        chunk0 = wid * n_chunks
        pltpu.sync_copy(idx_hbm.at[wid], idx_v)

        @pl.loop(0, n_chunks)
        def _(j):
            pltpu.async_copy(table_hbm.at[idx_v.at[j]], buf, sem).wait()
            pltpu.sync_copy(buf, out_hbm.at[pl.ds((chunk0 + j) * _CHUNK, _CHUNK)])

    return k


def kernel(indices, table):
    b, h = indices.shape
    _, d = table.shape
    n = b * h
    idx = indices.reshape(_NW, n // (_NW * _CHUNK), _CHUNK).astype(jnp.int32)
    out = _make_kernel(n, d)(idx, table)
    return out.reshape(b, h, d)


# speedup vs baseline: 4.6879x; 1.1469x over previous
import functools

import jax
import jax.numpy as jnp
from jax import lax
from jax.experimental import pallas as pl
from jax.experimental.pallas import tpu as pltpu
from jax.experimental.pallas import tpu_sc as plsc

_NC, _NS = 2, 16
_NW = _NC * _NS
_CHUNK = 128
_RING = 5


@functools.cache
def _make_kernel(n_rows: int, d: int):
    rows_per_w = n_rows // _NW
    n_chunks = rows_per_w // _CHUNK
    assert n_chunks % _RING == 0
    mesh = plsc.VectorSubcoreMesh(
        core_axis_name="c", subcore_axis_name="s",
        num_cores=_NC, num_subcores=_NS,
    )

    @functools.partial(
        pl.kernel,
        out_type=jax.ShapeDtypeStruct((n_rows, d), jnp.float32),
        mesh=mesh,
        scratch_types=[
            pltpu.VMEM((n_chunks, _CHUNK), jnp.int32),
            pltpu.VMEM((_RING, _CHUNK, d), jnp.float32),
        ] + [pltpu.SemaphoreType.DMA] * (2 * _RING),
        compiler_params=pltpu.CompilerParams(use_tc_tiling_on_sc=False),
    )
    def k(idx_hbm, table_hbm, out_hbm, idx_v, bufs, *sems):
        gsem, wsem = sems[:_RING], sems[_RING:]
        wid = lax.axis_index("s") * _NC + lax.axis_index("c")
        chunk0 = wid * n_chunks
        pltpu.sync_copy(idx_hbm.at[wid], idx_v)

        for b in range(_RING):
            pltpu.async_copy(table_hbm.at[idx_v.at[b]], bufs.at[b], gsem[b])

        @pl.loop(0, n_chunks, step=_RING)
        def _(j0):
            for b in range(_RING):
                j = j0 + b
                pltpu.make_async_copy(
                    table_hbm.at[idx_v.at[j]], bufs.at[b], gsem[b]).wait()
                out_slice = out_hbm.at[pl.ds((chunk0 + j) * _CHUNK, _CHUNK)]
                pltpu.async_copy(bufs.at[b], out_slice, wsem[b])
                j2 = j + _RING

                @pl.when(j2 < n_chunks)
                def _():
                    pltpu.make_async_copy(bufs.at[b], out_slice, wsem[b]).wait()
                    pltpu.async_copy(
                        table_hbm.at[idx_v.at[j2]], bufs.at[b], gsem[b])

        for b in range(_RING):
            j = n_chunks - _RING + b
            out_slice = out_hbm.at[pl.ds((chunk0 + j) * _CHUNK, _CHUNK)]
            pltpu.make_async_copy(bufs.at[b], out_slice, wsem[b]).wait()

    return k


def kernel(indices, table):
    b, h = indices.shape
    _, d = table.shape
    n = b * h
    idx = indices.reshape(_NW, n // (_NW * _CHUNK), _CHUNK).astype(jnp.int32)
    out = _make_kernel(n, d)(idx, table)
    return out.reshape(b, h, d)
